# baseline (speedup 1.0000x reference)
import functools

import jax
import jax.numpy as jnp
from jax import lax
from jax.experimental import pallas as pl
from jax.experimental.pallas import tpu as pltpu
from jax.experimental.pallas import tpu_sc as plsc

_BATCH = 4
_SEQ = 2048
_DIM = 128
_N = _BATCH * _SEQ
_NC = 2
_NS = 16
_NW = _NC * _NS
_BPW = _N // _NW
_W = 128
_NWIN = _BPW // _W
_SCALE = 11.313708498984761


def _sc_gather(idx_flat, table):
    mesh = plsc.VectorSubcoreMesh(core_axis_name="c", subcore_axis_name="s")

    @functools.partial(
        pl.kernel,
        mesh=mesh,
        out_type=jax.ShapeDtypeStruct((_N, _DIM), jnp.float32),
        scratch_types=[
            pltpu.VMEM((_BPW,), jnp.int32),
            pltpu.VMEM((_NWIN, _W, _DIM), jnp.float32),
            pltpu.SemaphoreType.DMA,
            pltpu.SemaphoreType.DMA,
            pltpu.SemaphoreType.DMA,
        ],
    )
    def gather_kernel(idx_hbm, table_hbm, out_hbm, idx_v, rows_v, isem, gsem, osem):
        wid = lax.axis_index("s") * _NC + lax.axis_index("c")
        base = wid * _BPW
        pltpu.async_copy(idx_hbm.at[pl.ds(base, _BPW)], idx_v, isem).wait()
        gathers = []
        for w in range(_NWIN):
            gathers.append(
                pltpu.async_copy(
                    table_hbm.at[idx_v.at[pl.ds(w * _W, _W)]], rows_v.at[w], gsem
                )
            )
        writes = []
        for w in range(_NWIN):
            gathers[w].wait()
            writes.append(
                pltpu.async_copy(
                    rows_v.at[w], out_hbm.at[pl.ds(base + w * _W, _W)], osem
                )
            )
        for cp in writes:
            cp.wait()

    return gather_kernel(idx_flat, table)


def _tc_finish(emb, pos_enc):
    def body(emb_ref, pos_ref, o_ref):
        o_ref[...] = emb_ref[...] * _SCALE + pos_ref[...][None, :, :]

    return pl.pallas_call(
        body,
        out_shape=jax.ShapeDtypeStruct((_BATCH, _SEQ, _DIM), jnp.float32),
    )(emb, pos_enc)


def kernel(x, table, pos_enc):
    idx = x.reshape(-1).astype(jnp.int32)
    emb = _sc_gather(idx, table).reshape(_BATCH, _SEQ, _DIM)
    return _tc_finish(emb, pos_enc)


# speedup vs baseline: 1.0586x; 1.0586x over previous
import functools

import jax
import jax.numpy as jnp
from jax import lax
from jax.experimental import pallas as pl
from jax.experimental.pallas import tpu as pltpu
from jax.experimental.pallas import tpu_sc as plsc

_BATCH = 4
_SEQ = 2048
_DIM = 128
_N = _BATCH * _SEQ
_NC = 2
_NS = 16
_NW = _NC * _NS
_BPW = _N // _NW
_W = 128
_NWIN = _BPW // _W
_WPB = _SEQ // _BPW
_SCALE = 11.313708498984761


def _sc_embed(x, table, pos_enc):
    mesh = plsc.VectorSubcoreMesh(core_axis_name="c", subcore_axis_name="s")

    @functools.partial(
        pl.kernel,
        mesh=mesh,
        out_type=jax.ShapeDtypeStruct((_BATCH, _SEQ, _DIM), jnp.float32),
        scratch_types=[
            pltpu.VMEM((_BPW,), jnp.int32),
            pltpu.VMEM((_NWIN, _W, _DIM), jnp.float32),
            pltpu.VMEM((_NWIN, _W, _DIM), jnp.float32),
            pltpu.SemaphoreType.DMA,
            pltpu.SemaphoreType.DMA,
            pltpu.SemaphoreType.DMA,
            pltpu.SemaphoreType.DMA,
        ],
    )
    def embed_kernel(idx_hbm, table_hbm, pos_hbm, out_hbm,
                     idx_v, rows_v, pos_v, isem, psem, gsem, osem):
        wid = lax.axis_index("s") * _NC + lax.axis_index("c")
        b = wid // _WPB
        s_off = (wid % _WPB) * _BPW
        icp = pltpu.async_copy(idx_hbm.at[b, pl.ds(s_off, _BPW)], idx_v, isem)
        pcps = [
            pltpu.async_copy(
                pos_hbm.at[pl.ds(s_off + w * _W, _W)], pos_v.at[w], psem
            )
            for w in range(_NWIN)
        ]
        icp.wait()
        gcps = [
            pltpu.async_copy(
                table_hbm.at[idx_v.at[pl.ds(w * _W, _W)]], rows_v.at[w], gsem
            )
            for w in range(_NWIN)
        ]
        ocps = []
        for w in range(_NWIN):
            gcps[w].wait()
            pcps[w].wait()

            @plsc.parallel_loop(0, _W, unroll=2)
            def _scale_add(r, w=w):
                for j in range(_DIM // 16):
                    sl = pl.ds(j * 16, 16)
                    rows_v[w, r, sl] = rows_v[w, r, sl] * _SCALE + pos_v[w, r, sl]

            ocps.append(
                pltpu.async_copy(
                    rows_v.at[w], out_hbm.at[b, pl.ds(s_off + w * _W, _W)], osem
                )
            )
        for cp in ocps:
            cp.wait()

    return embed_kernel(x, table, pos_enc)


def kernel(x, table, pos_enc):
    return _sc_embed(x.astype(jnp.int32), table, pos_enc)
